# baseline (speedup 1.0000x reference)
import jax
import jax.numpy as jnp
from jax.experimental import pallas as pl
from jax.experimental.pallas import tpu as pltpu

_LAM = 0.5
_TGT = 1.0 - _LAM
_GRP = 32
_N_BISECT = 8
_N_NEWTON_G = 4
_N_NEWTON_F = 1


def _sparsegen_rows_kernel(x_ref, o_ref):
    z = x_ref[:]
    r, n = z.shape
    g = n // _GRP
    s0 = z[:, :g]
    s1 = z[:, g:2 * g]
    m1 = jnp.maximum(s0, s1)
    m2 = jnp.minimum(s0, s1)
    for j in range(2, _GRP):
        sj = z[:, j * g:(j + 1) * g]
        m2 = jnp.maximum(m2, jnp.minimum(m1, sj))
        m1 = jnp.maximum(m1, sj)
    m = jnp.max(m1, axis=-1, keepdims=True)
    lo0 = m - _TGT
    wg = jnp.concatenate(
        [jnp.maximum(m1 - lo0, 0.0), jnp.maximum(m2 - lo0, 0.0)], axis=-1)

    def bisect_body(_, carry):
        dlo, dhi = carry
        mid = 0.5 * (dlo + dhi)
        f = jnp.sum(jnp.maximum(wg - mid, 0.0), axis=-1, keepdims=True)
        gt = f > _TGT
        return jnp.where(gt, mid, dlo), jnp.where(gt, dhi, mid)

    dlo0 = jnp.zeros_like(m)
    dhi0 = jnp.full_like(m, _TGT)
    dlo, _ = jax.lax.fori_loop(0, _N_BISECT, bisect_body, (dlo0, dhi0))

    def newton_g_body(_, d):
        mask = wg > d
        k = jnp.sum(mask.astype(z.dtype), axis=-1, keepdims=True)
        s = jnp.sum(jnp.where(mask, wg, 0.0), axis=-1, keepdims=True)
        return (s - _TGT) / jnp.maximum(k, 1.0)

    d = jax.lax.fori_loop(0, _N_NEWTON_G, newton_g_body, dlo)
    t0 = lo0 + d

    def newton_f_body(_, t):
        mask = z > t
        k = jnp.sum(mask.astype(z.dtype), axis=-1, keepdims=True)
        s = jnp.sum(jnp.where(mask, z, 0.0), axis=-1, keepdims=True)
        return (s - _TGT) / jnp.maximum(k, 1.0)

    tau = jax.lax.fori_loop(0, _N_NEWTON_F, newton_f_body, t0)

    o_ref[:] = jnp.maximum(z * (1.0 / _TGT) - tau * (1.0 / _TGT), 0.0)


@jax.jit
def kernel(input):
    b, q, n = input.shape
    rows = b * q
    x2 = input.reshape(rows, n)
    block_rows = 32
    out = pl.pallas_call(
        _sparsegen_rows_kernel,
        out_shape=jax.ShapeDtypeStruct((rows, n), input.dtype),
        grid=(rows // block_rows,),
        in_specs=[pl.BlockSpec((block_rows, n), lambda i: (i, 0))],
        out_specs=pl.BlockSpec((block_rows, n), lambda i: (i, 0)),
        compiler_params=pltpu.CompilerParams(
            dimension_semantics=("parallel",)),
    )(x2)
    return out.reshape(b, q, n)


# speedup vs baseline: 1.0048x; 1.0048x over previous
import jax
import jax.numpy as jnp
from jax.experimental import pallas as pl
from jax.experimental.pallas import tpu as pltpu

_LAM = 0.5
_TGT = 1.0 - _LAM
_GRP = 32
_N_BISECT = 8
_N_NEWTON_G = 4
_N_NEWTON_F = 1


def _sparsegen_rows_kernel(x_ref, o_ref):
    z = x_ref[:]
    r, n = z.shape
    g = n // _GRP
    s0 = z[:, :g]
    s1 = z[:, g:2 * g]
    m1 = jnp.maximum(s0, s1)
    m2 = jnp.minimum(s0, s1)
    for j in range(2, _GRP):
        sj = z[:, j * g:(j + 1) * g]
        m2 = jnp.maximum(m2, jnp.minimum(m1, sj))
        m1 = jnp.maximum(m1, sj)
    m = jnp.max(m1, axis=-1, keepdims=True)
    lo0 = m - _TGT
    wg = jnp.concatenate(
        [jnp.maximum(m1 - lo0, 0.0), jnp.maximum(m2 - lo0, 0.0)], axis=-1)

    def bisect_body(_, carry):
        dlo, dhi = carry
        mid = 0.5 * (dlo + dhi)
        f = jnp.sum(jnp.maximum(wg - mid, 0.0), axis=-1, keepdims=True)
        gt = f > _TGT
        return jnp.where(gt, mid, dlo), jnp.where(gt, dhi, mid)

    dlo0 = jnp.zeros_like(m)
    dhi0 = jnp.full_like(m, _TGT)
    dlo, _ = jax.lax.fori_loop(0, _N_BISECT, bisect_body, (dlo0, dhi0))

    def newton_g_body(_, d):
        mask = wg > d
        k = jnp.sum(mask.astype(z.dtype), axis=-1, keepdims=True)
        s = jnp.sum(jnp.where(mask, wg, 0.0), axis=-1, keepdims=True)
        return (s - _TGT) / jnp.maximum(k, 1.0)

    d = jax.lax.fori_loop(0, _N_NEWTON_G, newton_g_body, dlo)
    t0 = lo0 + d

    def newton_f_body(_, t):
        mask = z > t
        k = jnp.sum(mask.astype(z.dtype), axis=-1, keepdims=True)
        s = jnp.sum(jnp.where(mask, z, 0.0), axis=-1, keepdims=True)
        return (s - _TGT) / jnp.maximum(k, 1.0)

    tau = jax.lax.fori_loop(0, _N_NEWTON_F, newton_f_body, t0)

    o_ref[:] = jnp.maximum(z * (1.0 / _TGT) - tau * (1.0 / _TGT), 0.0)


@jax.jit
def kernel(input):
    b, q, n = input.shape
    rows = b * q
    x2 = input.reshape(rows, n)
    block_rows = 64
    out = pl.pallas_call(
        _sparsegen_rows_kernel,
        out_shape=jax.ShapeDtypeStruct((rows, n), input.dtype),
        grid=(rows // block_rows,),
        in_specs=[pl.BlockSpec((block_rows, n), lambda i: (i, 0))],
        out_specs=pl.BlockSpec((block_rows, n), lambda i: (i, 0)),
        compiler_params=pltpu.CompilerParams(
            dimension_semantics=("parallel",)),
    )(x2)
    return out.reshape(b, q, n)


# speedup vs baseline: 1.0715x; 1.0663x over previous
import jax
import jax.numpy as jnp
from jax.experimental import pallas as pl
from jax.experimental.pallas import tpu as pltpu

_LAM = 0.5
_TGT = 1.0 - _LAM
_GRP = 32
_N_BISECT = 8
_N_NEWTON_G = 5


def _sparsegen_rows_kernel(x_ref, o_ref):
    z = x_ref[:]
    r, n = z.shape
    g = n // _GRP
    s0 = z[:, :g]
    s1 = z[:, g:2 * g]
    m1 = jnp.maximum(s0, s1)
    m2 = jnp.minimum(s0, s1)
    m3 = jnp.full_like(m1, -jnp.inf)
    for j in range(2, _GRP):
        sj = z[:, j * g:(j + 1) * g]
        b1 = jnp.minimum(m1, sj)
        m1 = jnp.maximum(m1, sj)
        b2 = jnp.minimum(m2, b1)
        m2 = jnp.maximum(m2, b1)
        m3 = jnp.maximum(m3, b2)
    m = jnp.max(m1, axis=-1, keepdims=True)
    lo0 = m - _TGT
    wg = jnp.concatenate(
        [jnp.maximum(m1 - lo0, 0.0), jnp.maximum(m2 - lo0, 0.0),
         jnp.maximum(m3 - lo0, 0.0)], axis=-1)

    def bisect_body(_, carry):
        dlo, dhi = carry
        mid = 0.5 * (dlo + dhi)
        f = jnp.sum(jnp.maximum(wg - mid, 0.0), axis=-1, keepdims=True)
        gt = f > _TGT
        return jnp.where(gt, mid, dlo), jnp.where(gt, dhi, mid)

    dlo0 = jnp.zeros_like(m)
    dhi0 = jnp.full_like(m, _TGT)
    dlo, _ = jax.lax.fori_loop(0, _N_BISECT, bisect_body, (dlo0, dhi0))

    def newton_g_body(_, d):
        mask = wg > d
        k = jnp.sum(mask.astype(z.dtype), axis=-1, keepdims=True)
        s = jnp.sum(jnp.where(mask, wg, 0.0), axis=-1, keepdims=True)
        return (s - _TGT) / jnp.maximum(k, 1.0)

    d = jax.lax.fori_loop(0, _N_NEWTON_G, newton_g_body, dlo)
    tau = lo0 + d

    o_ref[:] = jnp.maximum(z * (1.0 / _TGT) - tau * (1.0 / _TGT), 0.0)


@jax.jit
def kernel(input):
    b, q, n = input.shape
    rows = b * q
    x2 = input.reshape(rows, n)
    block_rows = 64
    out = pl.pallas_call(
        _sparsegen_rows_kernel,
        out_shape=jax.ShapeDtypeStruct((rows, n), input.dtype),
        grid=(rows // block_rows,),
        in_specs=[pl.BlockSpec((block_rows, n), lambda i: (i, 0))],
        out_specs=pl.BlockSpec((block_rows, n), lambda i: (i, 0)),
        compiler_params=pltpu.CompilerParams(
            dimension_semantics=("parallel",)),
    )(x2)
    return out.reshape(b, q, n)


# speedup vs baseline: 1.1759x; 1.0975x over previous
import jax
import jax.numpy as jnp
from jax.experimental import pallas as pl
from jax.experimental.pallas import tpu as pltpu

_LAM = 0.5
_TGT = 1.0 - _LAM
_GRP = 64
_N_BISECT = 8
_N_NEWTON_G = 5


def _sparsegen_rows_kernel(x_ref, o_ref):
    z = x_ref[:]
    r, n = z.shape
    g = n // _GRP
    s0 = z[:, :g]
    s1 = z[:, g:2 * g]
    m1 = jnp.maximum(s0, s1)
    m2 = jnp.minimum(s0, s1)
    m3 = jnp.full_like(m1, -jnp.inf)
    for j in range(2, _GRP):
        sj = z[:, j * g:(j + 1) * g]
        b1 = jnp.minimum(m1, sj)
        m1 = jnp.maximum(m1, sj)
        b2 = jnp.minimum(m2, b1)
        m2 = jnp.maximum(m2, b1)
        m3 = jnp.maximum(m3, b2)
    m = jnp.max(m1, axis=-1, keepdims=True)
    lo0 = m - _TGT
    wg = jnp.concatenate(
        [jnp.maximum(m1 - lo0, 0.0), jnp.maximum(m2 - lo0, 0.0),
         jnp.maximum(m3 - lo0, 0.0)], axis=-1)

    def bisect_body(_, carry):
        dlo, dhi = carry
        mid = 0.5 * (dlo + dhi)
        f = jnp.sum(jnp.maximum(wg - mid, 0.0), axis=-1, keepdims=True)
        gt = f > _TGT
        return jnp.where(gt, mid, dlo), jnp.where(gt, dhi, mid)

    dlo0 = jnp.zeros_like(m)
    dhi0 = jnp.full_like(m, _TGT)
    dlo, _ = jax.lax.fori_loop(0, _N_BISECT, bisect_body, (dlo0, dhi0))

    def newton_g_body(_, d):
        mask = wg > d
        k = jnp.sum(mask.astype(z.dtype), axis=-1, keepdims=True)
        s = jnp.sum(jnp.where(mask, wg, 0.0), axis=-1, keepdims=True)
        return (s - _TGT) / jnp.maximum(k, 1.0)

    d = jax.lax.fori_loop(0, _N_NEWTON_G, newton_g_body, dlo)
    tau = lo0 + d

    o_ref[:] = jnp.maximum(z * (1.0 / _TGT) - tau * (1.0 / _TGT), 0.0)


@jax.jit
def kernel(input):
    b, q, n = input.shape
    rows = b * q
    x2 = input.reshape(rows, n)
    block_rows = 64
    out = pl.pallas_call(
        _sparsegen_rows_kernel,
        out_shape=jax.ShapeDtypeStruct((rows, n), input.dtype),
        grid=(rows // block_rows,),
        in_specs=[pl.BlockSpec((block_rows, n), lambda i: (i, 0))],
        out_specs=pl.BlockSpec((block_rows, n), lambda i: (i, 0)),
        compiler_params=pltpu.CompilerParams(
            dimension_semantics=("parallel",)),
    )(x2)
    return out.reshape(b, q, n)


# speedup vs baseline: 1.2995x; 1.1051x over previous
import jax
import jax.numpy as jnp
from jax.experimental import pallas as pl
from jax.experimental.pallas import tpu as pltpu

_LAM = 0.5
_TGT = 1.0 - _LAM
_GRP = 128
_N_BISECT = 8
_N_NEWTON_G = 5


def _sparsegen_rows_kernel(x_ref, o_ref):
    z = x_ref[:]
    r, n = z.shape
    g = n // _GRP
    s0 = z[:, :g]
    s1 = z[:, g:2 * g]
    m1 = jnp.maximum(s0, s1)
    m2 = jnp.minimum(s0, s1)
    m3 = jnp.full_like(m1, -jnp.inf)
    for j in range(2, _GRP):
        sj = z[:, j * g:(j + 1) * g]
        b1 = jnp.minimum(m1, sj)
        m1 = jnp.maximum(m1, sj)
        b2 = jnp.minimum(m2, b1)
        m2 = jnp.maximum(m2, b1)
        m3 = jnp.maximum(m3, b2)
    m = jnp.max(m1, axis=-1, keepdims=True)
    lo0 = m - _TGT
    wg = jnp.concatenate(
        [jnp.maximum(m1 - lo0, 0.0), jnp.maximum(m2 - lo0, 0.0),
         jnp.maximum(m3 - lo0, 0.0)], axis=-1)

    def bisect_body(_, carry):
        dlo, dhi = carry
        mid = 0.5 * (dlo + dhi)
        f = jnp.sum(jnp.maximum(wg - mid, 0.0), axis=-1, keepdims=True)
        gt = f > _TGT
        return jnp.where(gt, mid, dlo), jnp.where(gt, dhi, mid)

    dlo0 = jnp.zeros_like(m)
    dhi0 = jnp.full_like(m, _TGT)
    dlo, _ = jax.lax.fori_loop(0, _N_BISECT, bisect_body, (dlo0, dhi0))

    def newton_g_body(_, d):
        mask = wg > d
        k = jnp.sum(mask.astype(z.dtype), axis=-1, keepdims=True)
        s = jnp.sum(jnp.where(mask, wg, 0.0), axis=-1, keepdims=True)
        return (s - _TGT) / jnp.maximum(k, 1.0)

    d = jax.lax.fori_loop(0, _N_NEWTON_G, newton_g_body, dlo)
    tau = lo0 + d

    o_ref[:] = jnp.maximum(z * (1.0 / _TGT) - tau * (1.0 / _TGT), 0.0)


@jax.jit
def kernel(input):
    b, q, n = input.shape
    rows = b * q
    x2 = input.reshape(rows, n)
    block_rows = 64
    out = pl.pallas_call(
        _sparsegen_rows_kernel,
        out_shape=jax.ShapeDtypeStruct((rows, n), input.dtype),
        grid=(rows // block_rows,),
        in_specs=[pl.BlockSpec((block_rows, n), lambda i: (i, 0))],
        out_specs=pl.BlockSpec((block_rows, n), lambda i: (i, 0)),
        compiler_params=pltpu.CompilerParams(
            dimension_semantics=("parallel",)),
    )(x2)
    return out.reshape(b, q, n)
